# baseline (speedup 1.0000x reference)
import functools

import jax
import jax.numpy as jnp
from jax import lax
from jax.experimental import pallas as pl
from jax.experimental.pallas import tpu as pltpu
from jax.experimental.pallas import tpu_sc as plsc

N = 10000
NPAD = 10240
E = 160000
EPAD = 163840
D = 128
HD = 64
NREL = 8
B = 1024
BINROWS = 640
EPW = EPAD // 16
NCH = EPW // 128
EPW32 = EPAD // 32

_MESH = plsc.VectorSubcoreMesh(core_axis_name="c", subcore_axis_name="s")


def _z16():
    return jnp.zeros((16,), jnp.float32)


def _one16():
    return jnp.ones((16,), jnp.float32)


NBINS = 81920


@functools.partial(
    pl.kernel,
    out_type=(jax.ShapeDtypeStruct((2 * EPAD,), jnp.float32),
              jax.ShapeDtypeStruct((2 * EPAD,), jnp.float32)),
    mesh=_MESH,
    compiler_params=pltpu.CompilerParams(needs_layout_passes=False),
    scratch_types=[
        pltpu.VMEM((BINROWS, 128), jnp.float32),
        pltpu.VMEM((2560,), jnp.int32),
        pltpu.VMEM((2560,), jnp.int32),
        pltpu.VMEM((5, 128), jnp.int32),
        pltpu.VMEM((2 * 2560,), jnp.float32),
        pltpu.VMEM((32,), jnp.float32),
        pltpu.VMEM_SHARED((BINROWS, 128), jnp.float32),
    ],
)
def _stage1(dst_hbm, et_hbm, comp_hbm, fm1_hbm, fm2_hbm,
            counts_v, d_v, e_v, ridx_v, co_v, comp_v, tot_sh):
    c = lax.axis_index("c")
    s = lax.axis_index("s")

    def _zrow(i, _):
        for k in range(8):
            counts_v[i, pl.ds(k * 16, 16)] = _z16()
        return 0
    lax.fori_loop(0, BINROWS, _zrow, 0)
    pltpu.sync_copy(comp_hbm, comp_v)

    @pl.when(s == 0)
    def _():
        pltpu.sync_copy(counts_v, tot_sh)

    def _ridx(i, _):
        for k in range(8):
            ridx_v[i, pl.ds(k * 16, 16)] = i * 128 + k * 16 + lax.iota(jnp.int32, 16)
        return 0
    lax.fori_loop(0, 5, _ridx, 0)

    def _hist_chunk(ci, _):
        base = s * EPW + ci * 2560
        pltpu.sync_copy(dst_hbm.at[pl.ds(base, 2560)], d_v)
        pltpu.sync_copy(et_hbm.at[pl.ds(base, 2560)], e_v)

        def _cnt(i, _2):
            for k in range(4):
                o = i * 64 + k * 16
                key = d_v[pl.ds(o, 16)] * NREL + e_v[pl.ds(o, 16)]
                row = lax.shift_right_logical(key, 7)
                col = lax.bitwise_and(key, 127)
                plsc.addupdate_scatter(counts_v, [row, col], _one16())
            return 0
        lax.fori_loop(0, 40, _cnt, 0)
        return 0
    lax.fori_loop(0, EPW // 2560, _hist_chunk, 0)

    plsc.subcore_barrier()
    for j in range(5):
        pltpu.sync_copy(counts_v.at[pl.ds(j * 128, 128)],
                        tot_sh.at[ridx_v.at[j]], add=True)
    plsc.subcore_barrier()
    pltpu.sync_copy(tot_sh, counts_v)

    w = c * 16 + s

    def _co_chunk(ci, _):
        base = w * EPW32 + ci * 2560
        pltpu.sync_copy(dst_hbm.at[pl.ds(base, 2560)], d_v)
        pltpu.sync_copy(et_hbm.at[pl.ds(base, 2560)], e_v)

        def _co(i, _2):
            o2 = i * 16
            ev = e_v[pl.ds(o2, 16)]
            key = d_v[pl.ds(o2, 16)] * NREL + ev
            row = lax.shift_right_logical(key, 7)
            col = lax.bitwise_and(key, 127)
            cnt = plsc.load_gather(counts_v, [row, col])
            nrm = _one16() / jnp.maximum(cnt, 1.0)
            q = lax.shift_right_logical(i, 3)
            rem = lax.bitwise_and(i, 7) * 16
            for lb in range(2):
                cm = plsc.load_gather(comp_v, [ev + lb * NREL])
                co_v[pl.ds(q * 256 + lb * 128 + rem, 16)] = cm * nrm
            return 0
        lax.fori_loop(0, 160, _co, 0)
        pltpu.sync_copy(co_v,
                        fm1_hbm.at[pl.ds(w * 2 * EPW32 + ci * 5120, 5120)])

        def _co2(i, _2):
            o2 = i * 16
            ev = e_v[pl.ds(o2, 16)]
            key = d_v[pl.ds(o2, 16)] * NREL + ev
            row = lax.shift_right_logical(key, 7)
            col = lax.bitwise_and(key, 127)
            cnt = plsc.load_gather(counts_v, [row, col])
            nrm = _one16() / jnp.maximum(cnt, 1.0)
            q = lax.shift_right_logical(i, 3)
            rem = lax.bitwise_and(i, 7) * 16
            for lb in range(2):
                cm = plsc.load_gather(comp_v, [ev + (lb + 2) * NREL])
                co_v[pl.ds(q * 256 + lb * 128 + rem, 16)] = cm * nrm
            return 0
        lax.fori_loop(0, 160, _co2, 0)
        pltpu.sync_copy(co_v,
                        fm2_hbm.at[pl.ds(w * 2 * EPW32 + ci * 5120, 5120)])
        return 0
    lax.fori_loop(0, 2, _co_chunk, 0)


NBLK = NCH // 8


@functools.partial(
    pl.kernel,
    out_type=jax.ShapeDtypeStruct((2 * NPAD, 128), jnp.float32),
    mesh=_MESH,
    compiler_params=pltpu.CompilerParams(needs_layout_passes=False,
                                         use_tc_tiling_on_sc=False),
    scratch_types=[
        pltpu.VMEM((8, 5, 128), jnp.int32),
        pltpu.VMEM((8, 5, 128), jnp.int32),
        pltpu.VMEM((128, HD), jnp.float32),
        pltpu.VMEM((128, HD), jnp.float32),
        pltpu.VMEM((128, 128), jnp.float32),
        pltpu.VMEM_SHARED((NPAD, 128), jnp.float32),
        pltpu.SemaphoreType.DMA,
        pltpu.SemaphoreType.DMA,
        pltpu.SemaphoreType.DMA,
        pltpu.SemaphoreType.DMA,
    ],
)
def _edge_pass(xh_hbm, meta_hbm, cc_hbm,
               mblk_a, mblk_b, rows_a, rows_b, out_v, csh,
               msem_a, msem_b, gsem_a, gsem_b):
    c = lax.axis_index("c")
    s = lax.axis_index("s")
    mblk = (mblk_a, mblk_b)
    rows = (rows_a, rows_b)
    msem = (msem_a, msem_b)
    gsem = (gsem_a, gsem_b)

    def _z(i, _):
        for k in range(8):
            out_v[i, pl.ds(k * 16, 16)] = _z16()
        return 0
    lax.fori_loop(0, 128, _z, 0)
    rows_per_w = NPAD // 16
    for j in range(rows_per_w // 128):
        pltpu.sync_copy(out_v, csh.at[pl.ds(s * rows_per_w + j * 128, 128)])
    plsc.subcore_barrier()

    def _meta_start(kb, m):
        pltpu.async_copy(meta_hbm.at[s, kb], mblk[m], msem[m])

    def _meta_wait(kb, m):
        pltpu.make_async_copy(meta_hbm.at[s, kb], mblk[m], msem[m]).wait()

    def _gl(m, q8, pe):
        pltpu.async_copy(xh_hbm.at[mblk[m].at[q8, c]], rows[pe], gsem[pe])

    def _gw(m, q8, pe):
        pltpu.make_async_copy(xh_hbm.at[mblk[m].at[q8, c]], rows[pe],
                              gsem[pe]).wait()

    def _compute_scatter(m, q8, pe):
        def _grp(g, _g):
            c0g = plsc.bitcast(mblk[m][q8, 3, pl.ds(g * 16, 16)], jnp.float32)
            c1g = plsc.bitcast(mblk[m][q8, 4, pl.ds(g * 16, 16)], jnp.float32)
            for k in range(16):
                r = g * 16 + k
                c0s = jnp.broadcast_to(c0g[k], (16,))
                c1s = jnp.broadcast_to(c1g[k], (16,))
                for h in range(HD // 16):
                    rv = rows[pe][r, pl.ds(h * 16, 16)]
                    out_v[r, pl.ds(h * 16, 16)] = rv * c0s
                    out_v[r, pl.ds(HD + h * 16, 16)] = rv * c1s
            return 0
        lax.fori_loop(0, 8, _grp, 0)
        pltpu.sync_copy(out_v, csh.at[mblk[m].at[q8, 2]], add=True)

    _meta_start(0, 0)
    _meta_wait(0, 0)
    _meta_start(1, 1)
    _gl(0, 0, 0)

    def _block_pair(jj, _):
        for mb in range(2):
            kb = jj * 2 + mb
            m = mb
            m1 = 1 - mb

            def _pair(pr, _p):
                for pe in range(2):
                    q8 = pr * 2 + pe

                    @pl.when(q8 < 7)
                    def _():
                        _gl(m, q8 + 1, 1 - pe)

                    @pl.when((q8 == 7) & (kb < NBLK - 1))
                    def _():
                        _meta_wait(kb + 1, m1)
                        _gl(m1, 0, 1 - pe)

                    _gw(m, q8, pe)
                    _compute_scatter(m, q8, pe)

                    @pl.when((q8 == 7) & (kb < NBLK - 2))
                    def _():
                        _meta_start(kb + 2, m)
                return 0
            lax.fori_loop(0, 4, _pair, 0)
        return 0
    lax.fori_loop(0, NBLK // 2, _block_pair, 0)

    plsc.subcore_barrier()
    pltpu.sync_copy(csh.at[pl.ds(s * rows_per_w, rows_per_w)],
                    cc_hbm.at[pl.ds(c * NPAD + s * rows_per_w, rows_per_w)])


def _dense_body(c0_ref, c1_ref, x_ref, w0_ref, w1_ref, wr_ref, b_ref, o_ref,
                *, relu):
    hp = jax.lax.Precision.HIGHEST
    acc = jnp.dot(c0_ref[...], w0_ref[...], precision=hp,
                  preferred_element_type=jnp.float32)
    acc = acc + jnp.dot(c1_ref[...], w1_ref[...], precision=hp,
                        preferred_element_type=jnp.float32)
    acc = acc + jnp.dot(x_ref[...], wr_ref[...], precision=hp,
                        preferred_element_type=jnp.float32)
    acc = acc + b_ref[...]
    if relu:
        acc = jnp.maximum(acc, 0.0)
    o_ref[...] = acc


def _dense(c0, c1, x, w0, w1, wr, bias2d, relu):
    blk = 1024
    rbs = lambda: pl.BlockSpec((blk, 128), lambda i: (i, 0))
    wbs = lambda: pl.BlockSpec((128, 128), lambda i: (0, 0))
    return pl.pallas_call(
        functools.partial(_dense_body, relu=relu),
        grid=(NPAD // blk,),
        in_specs=[rbs(), rbs(), rbs(), wbs(), wbs(), wbs(),
                  pl.BlockSpec((1, 128), lambda i: (0, 0))],
        out_specs=rbs(),
        out_shape=jax.ShapeDtypeStruct((NPAD, 128), jnp.float32),
    )(c0, c1, x, w0, w1, wr, bias2d)


@functools.partial(
    pl.kernel,
    out_type=jax.ShapeDtypeStruct((B,), jnp.float32),
    mesh=_MESH,
    compiler_params=pltpu.CompilerParams(needs_layout_passes=False),
    scratch_types=[
        pltpu.VMEM((32,), jnp.int32),
        pltpu.VMEM((32,), jnp.int32),
        pltpu.VMEM((32,), jnp.int32),
        pltpu.VMEM((32, 128), jnp.float32),
        pltpu.VMEM((32, 128), jnp.float32),
        pltpu.VMEM((32, 128), jnp.float32),
        pltpu.VMEM((32,), jnp.float32),
        pltpu.SemaphoreType.DMA,
        pltpu.SemaphoreType.DMA,
        pltpu.SemaphoreType.DMA,
    ],
)
def _score(h_hbm, rel_hbm, heads_hbm, rels_hbm, tails_hbm, out_hbm,
           hi_v, ti_v, ri_v, he_v, te_v, re_v, sc_v, sem0, sem1, sem2):
    c = lax.axis_index("c")
    s = lax.axis_index("s")
    w = c * 16 + s
    off = w * 32
    pltpu.sync_copy(heads_hbm.at[pl.ds(off, 32)], hi_v)
    pltpu.sync_copy(tails_hbm.at[pl.ds(off, 32)], ti_v)
    pltpu.sync_copy(rels_hbm.at[pl.ds(off, 32)], ri_v)
    cp0 = pltpu.async_copy(h_hbm.at[hi_v], he_v, sem0)
    cp1 = pltpu.async_copy(h_hbm.at[ti_v], te_v, sem1)
    cp2 = pltpu.async_copy(rel_hbm.at[ri_v], re_v, sem2)
    cp0.wait()
    cp1.wait()
    cp2.wait()
    iota16 = lax.iota(jnp.int32, 16)
    for grp in range(2):
        sv = _z16()
        for i in range(16):
            t = grp * 16 + i
            acc = _z16()
            for hh in range(8):
                acc = acc + (he_v[t, pl.ds(hh * 16, 16)]
                             * re_v[t, pl.ds(hh * 16, 16)]
                             * te_v[t, pl.ds(hh * 16, 16)])
            stot = jnp.sum(acc)
            sv = jnp.where(iota16 == i, jnp.broadcast_to(stot, (16,)), sv)
        sc_v[pl.ds(grp * 16, 16)] = sv
    pltpu.sync_copy(sc_v, out_hbm.at[pl.ds(off, 32)])


def kernel(heads, relations, tails, edge_index, edge_type, entity_emb,
           relation_emb, comp1, bases1, root1, bias1, comp2, bases2, root2,
           bias2):
    src = edge_index[0]
    dst = edge_index[1]
    padn = EPAD - E
    src_p = jnp.concatenate([src, jnp.zeros((padn,), jnp.int32)])
    dst_p = jnp.concatenate([dst, jnp.full((padn,), N, jnp.int32)])
    et_p = jnp.concatenate([edge_type, jnp.zeros((padn,), jnp.int32)])
    comp_cat = jnp.concatenate([comp1[:, 0], comp1[:, 1],
                                comp2[:, 0], comp2[:, 1]])

    fm1, fm2 = _stage1(dst_p, et_p, comp_cat)

    src3 = src_p.reshape(16, NCH, 128)
    dst3 = dst_p.reshape(16, NCH, 128)
    imeta = jnp.stack([2 * src3, 2 * src3 + 1, dst3], axis=2)
    meta = []
    for fm_l in (fm1, fm2):
        fi = lax.bitcast_convert_type(fm_l.reshape(16, NCH, 2, 128), jnp.int32)
        m5 = jnp.concatenate([imeta, fi], axis=2)
        meta.append(m5.reshape(16, NCH // 8, 8, 5, 128))
    xpad = jnp.pad(entity_emb, ((0, NPAD - N), (0, 0)))

    h = xpad
    layer_params = (
        (0, bases1, root1, bias1.reshape(1, D), True),
        (1, bases2, root2, bias2.reshape(1, D), False),
    )
    for l, bases, root, bias2d, relu in layer_params:
        xh = h.reshape(NPAD, 2, HD).reshape(2 * NPAD, HD)
        ccat = _edge_pass(xh, meta[l])
        w0 = jnp.concatenate([bases[0][:HD, :], bases[1][:HD, :]], axis=0)
        w1 = jnp.concatenate([bases[0][HD:, :], bases[1][HD:, :]], axis=0)
        h = _dense(ccat[:NPAD], ccat[NPAD:], h, w0, w1, root, bias2d, relu)

    return _score(h, relation_emb, heads, relations, tails)


# speedup vs baseline: 1.1514x; 1.1514x over previous
import functools

import jax
import jax.numpy as jnp
from jax import lax
from jax.experimental import pallas as pl
from jax.experimental.pallas import tpu as pltpu
from jax.experimental.pallas import tpu_sc as plsc

N = 10000
NPAD = 10240
E = 160000
EPAD = 163840
D = 128
HD = 64
NREL = 8
B = 1024
BINROWS = 640
EPW = EPAD // 16
NCH = EPW // 128
EPW32 = EPAD // 32

_MESH = plsc.VectorSubcoreMesh(core_axis_name="c", subcore_axis_name="s")


def _z16():
    return jnp.zeros((16,), jnp.float32)


def _one16():
    return jnp.ones((16,), jnp.float32)


NBINS = 81920


@functools.partial(
    pl.kernel,
    out_type=(jax.ShapeDtypeStruct((2 * EPAD,), jnp.float32),
              jax.ShapeDtypeStruct((2 * EPAD,), jnp.float32)),
    mesh=_MESH,
    compiler_params=pltpu.CompilerParams(needs_layout_passes=False),
    scratch_types=[
        pltpu.VMEM((BINROWS, 128), jnp.float32),
        pltpu.VMEM((2560,), jnp.int32),
        pltpu.VMEM((2560,), jnp.int32),
        pltpu.VMEM((5, 128), jnp.int32),
        pltpu.VMEM((2 * 2560,), jnp.float32),
        pltpu.VMEM((32,), jnp.float32),
        pltpu.VMEM_SHARED((BINROWS, 128), jnp.float32),
    ],
)
def _stage1(dst_hbm, et_hbm, comp_hbm, fm1_hbm, fm2_hbm,
            counts_v, d_v, e_v, ridx_v, co_v, comp_v, tot_sh):
    c = lax.axis_index("c")
    s = lax.axis_index("s")

    def _zrow(i, _):
        for k in range(8):
            counts_v[i, pl.ds(k * 16, 16)] = _z16()
        return 0
    lax.fori_loop(0, BINROWS, _zrow, 0)
    pltpu.sync_copy(comp_hbm, comp_v)

    @pl.when(s == 0)
    def _():
        pltpu.sync_copy(counts_v, tot_sh)

    def _ridx(i, _):
        for k in range(8):
            ridx_v[i, pl.ds(k * 16, 16)] = i * 128 + k * 16 + lax.iota(jnp.int32, 16)
        return 0
    lax.fori_loop(0, 5, _ridx, 0)

    def _hist_chunk(ci, _):
        base = s * EPW + ci * 2560
        pltpu.sync_copy(dst_hbm.at[pl.ds(base, 2560)], d_v)
        pltpu.sync_copy(et_hbm.at[pl.ds(base, 2560)], e_v)

        def _cnt(i, _2):
            for k in range(4):
                o = i * 64 + k * 16
                key = d_v[pl.ds(o, 16)] * NREL + e_v[pl.ds(o, 16)]
                row = lax.shift_right_logical(key, 7)
                col = lax.bitwise_and(key, 127)
                plsc.addupdate_scatter(counts_v, [row, col], _one16())
            return 0
        lax.fori_loop(0, 40, _cnt, 0)
        return 0
    lax.fori_loop(0, EPW // 2560, _hist_chunk, 0)

    plsc.subcore_barrier()
    for j in range(5):
        pltpu.sync_copy(counts_v.at[pl.ds(j * 128, 128)],
                        tot_sh.at[ridx_v.at[j]], add=True)
    plsc.subcore_barrier()
    pltpu.sync_copy(tot_sh, counts_v)

    w = c * 16 + s

    def _co_chunk(ci, _):
        base = w * EPW32 + ci * 2560
        pltpu.sync_copy(dst_hbm.at[pl.ds(base, 2560)], d_v)
        pltpu.sync_copy(et_hbm.at[pl.ds(base, 2560)], e_v)

        def _co(i, _2):
            o2 = i * 16
            ev = e_v[pl.ds(o2, 16)]
            key = d_v[pl.ds(o2, 16)] * NREL + ev
            row = lax.shift_right_logical(key, 7)
            col = lax.bitwise_and(key, 127)
            cnt = plsc.load_gather(counts_v, [row, col])
            nrm = _one16() / jnp.maximum(cnt, 1.0)
            q = lax.shift_right_logical(i, 3)
            rem = lax.bitwise_and(i, 7) * 16
            for lb in range(2):
                cm = plsc.load_gather(comp_v, [ev + lb * NREL])
                co_v[pl.ds(q * 256 + lb * 128 + rem, 16)] = cm * nrm
            return 0
        lax.fori_loop(0, 160, _co, 0)
        pltpu.sync_copy(co_v,
                        fm1_hbm.at[pl.ds(w * 2 * EPW32 + ci * 5120, 5120)])

        def _co2(i, _2):
            o2 = i * 16
            ev = e_v[pl.ds(o2, 16)]
            key = d_v[pl.ds(o2, 16)] * NREL + ev
            row = lax.shift_right_logical(key, 7)
            col = lax.bitwise_and(key, 127)
            cnt = plsc.load_gather(counts_v, [row, col])
            nrm = _one16() / jnp.maximum(cnt, 1.0)
            q = lax.shift_right_logical(i, 3)
            rem = lax.bitwise_and(i, 7) * 16
            for lb in range(2):
                cm = plsc.load_gather(comp_v, [ev + (lb + 2) * NREL])
                co_v[pl.ds(q * 256 + lb * 128 + rem, 16)] = cm * nrm
            return 0
        lax.fori_loop(0, 160, _co2, 0)
        pltpu.sync_copy(co_v,
                        fm2_hbm.at[pl.ds(w * 2 * EPW32 + ci * 5120, 5120)])
        return 0
    lax.fori_loop(0, 2, _co_chunk, 0)


NBLK = NCH // 8


@functools.partial(
    pl.kernel,
    out_type=jax.ShapeDtypeStruct((2 * NPAD, 128), jnp.float32),
    mesh=_MESH,
    compiler_params=pltpu.CompilerParams(needs_layout_passes=False,
                                         use_tc_tiling_on_sc=False),
    scratch_types=[
        pltpu.VMEM((8, 5, 128), jnp.int32),
        pltpu.VMEM((8, 5, 128), jnp.int32),
        pltpu.VMEM((128, HD), jnp.float32),
        pltpu.VMEM((128, HD), jnp.float32),
        pltpu.VMEM((128, 128), jnp.float32),
        pltpu.VMEM_SHARED((NPAD, 128), jnp.float32),
        pltpu.SemaphoreType.DMA,
        pltpu.SemaphoreType.DMA,
        pltpu.SemaphoreType.DMA,
        pltpu.SemaphoreType.DMA,
    ],
)
def _edge_pass(xh_hbm, meta_hbm, cc_hbm,
               mblk_a, mblk_b, rows_a, rows_b, out_v, csh,
               msem_a, msem_b, gsem_a, gsem_b):
    c = lax.axis_index("c")
    s = lax.axis_index("s")
    mblk = (mblk_a, mblk_b)
    rows = (rows_a, rows_b)
    msem = (msem_a, msem_b)
    gsem = (gsem_a, gsem_b)

    def _z(i, _):
        for k in range(8):
            out_v[i, pl.ds(k * 16, 16)] = _z16()
        return 0
    lax.fori_loop(0, 128, _z, 0)
    rows_per_w = NPAD // 16
    for j in range(rows_per_w // 128):
        pltpu.sync_copy(out_v, csh.at[pl.ds(s * rows_per_w + j * 128, 128)])
    plsc.subcore_barrier()

    def _meta_start(kb, m):
        pltpu.async_copy(meta_hbm.at[s, kb], mblk[m], msem[m])

    def _meta_wait(kb, m):
        pltpu.make_async_copy(meta_hbm.at[s, kb], mblk[m], msem[m]).wait()

    def _gl(m, q8, pe):
        pltpu.async_copy(xh_hbm.at[mblk[m].at[q8, c]], rows[pe], gsem[pe])

    def _gw(m, q8, pe):
        pltpu.make_async_copy(xh_hbm.at[mblk[m].at[q8, c]], rows[pe],
                              gsem[pe]).wait()

    def _compute_scatter(m, q8, pe):
        @plsc.parallel_loop(0, 8)
        def _grp(g):
            c0g = plsc.bitcast(mblk[m][q8, 3, pl.ds(g * 16, 16)], jnp.float32)
            c1g = plsc.bitcast(mblk[m][q8, 4, pl.ds(g * 16, 16)], jnp.float32)
            for k in range(16):
                r = g * 16 + k
                c0s = jnp.broadcast_to(c0g[k], (16,))
                c1s = jnp.broadcast_to(c1g[k], (16,))
                for h in range(HD // 16):
                    rv = rows[pe][r, pl.ds(h * 16, 16)]
                    out_v[r, pl.ds(h * 16, 16)] = rv * c0s
                    out_v[r, pl.ds(HD + h * 16, 16)] = rv * c1s
        pltpu.sync_copy(out_v, csh.at[mblk[m].at[q8, 2]], add=True)

    _meta_start(0, 0)
    _meta_wait(0, 0)
    _meta_start(1, 1)
    _gl(0, 0, 0)

    def _block_pair(jj, _):
        for mb in range(2):
            kb = jj * 2 + mb
            m = mb
            m1 = 1 - mb

            def _pair(pr, _p):
                for pe in range(2):
                    q8 = pr * 2 + pe

                    @pl.when(q8 < 7)
                    def _():
                        _gl(m, q8 + 1, 1 - pe)

                    @pl.when((q8 == 7) & (kb < NBLK - 1))
                    def _():
                        _meta_wait(kb + 1, m1)
                        _gl(m1, 0, 1 - pe)

                    _gw(m, q8, pe)
                    _compute_scatter(m, q8, pe)

                    @pl.when((q8 == 7) & (kb < NBLK - 2))
                    def _():
                        _meta_start(kb + 2, m)
                return 0
            lax.fori_loop(0, 4, _pair, 0)
        return 0
    lax.fori_loop(0, NBLK // 2, _block_pair, 0)

    plsc.subcore_barrier()
    pltpu.sync_copy(csh.at[pl.ds(s * rows_per_w, rows_per_w)],
                    cc_hbm.at[pl.ds(c * NPAD + s * rows_per_w, rows_per_w)])


def _dense_body(c0_ref, c1_ref, x_ref, w0_ref, w1_ref, wr_ref, b_ref, o_ref,
                *, relu):
    hp = jax.lax.Precision.HIGHEST
    acc = jnp.dot(c0_ref[...], w0_ref[...], precision=hp,
                  preferred_element_type=jnp.float32)
    acc = acc + jnp.dot(c1_ref[...], w1_ref[...], precision=hp,
                        preferred_element_type=jnp.float32)
    acc = acc + jnp.dot(x_ref[...], wr_ref[...], precision=hp,
                        preferred_element_type=jnp.float32)
    acc = acc + b_ref[...]
    if relu:
        acc = jnp.maximum(acc, 0.0)
    o_ref[...] = acc


def _dense(c0, c1, x, w0, w1, wr, bias2d, relu):
    blk = 1024
    rbs = lambda: pl.BlockSpec((blk, 128), lambda i: (i, 0))
    wbs = lambda: pl.BlockSpec((128, 128), lambda i: (0, 0))
    return pl.pallas_call(
        functools.partial(_dense_body, relu=relu),
        grid=(NPAD // blk,),
        in_specs=[rbs(), rbs(), rbs(), wbs(), wbs(), wbs(),
                  pl.BlockSpec((1, 128), lambda i: (0, 0))],
        out_specs=rbs(),
        out_shape=jax.ShapeDtypeStruct((NPAD, 128), jnp.float32),
    )(c0, c1, x, w0, w1, wr, bias2d)


@functools.partial(
    pl.kernel,
    out_type=jax.ShapeDtypeStruct((B,), jnp.float32),
    mesh=_MESH,
    compiler_params=pltpu.CompilerParams(needs_layout_passes=False),
    scratch_types=[
        pltpu.VMEM((32,), jnp.int32),
        pltpu.VMEM((32,), jnp.int32),
        pltpu.VMEM((32,), jnp.int32),
        pltpu.VMEM((32, 128), jnp.float32),
        pltpu.VMEM((32, 128), jnp.float32),
        pltpu.VMEM((32, 128), jnp.float32),
        pltpu.VMEM((32,), jnp.float32),
        pltpu.SemaphoreType.DMA,
        pltpu.SemaphoreType.DMA,
        pltpu.SemaphoreType.DMA,
    ],
)
def _score(h_hbm, rel_hbm, heads_hbm, rels_hbm, tails_hbm, out_hbm,
           hi_v, ti_v, ri_v, he_v, te_v, re_v, sc_v, sem0, sem1, sem2):
    c = lax.axis_index("c")
    s = lax.axis_index("s")
    w = c * 16 + s
    off = w * 32
    pltpu.sync_copy(heads_hbm.at[pl.ds(off, 32)], hi_v)
    pltpu.sync_copy(tails_hbm.at[pl.ds(off, 32)], ti_v)
    pltpu.sync_copy(rels_hbm.at[pl.ds(off, 32)], ri_v)
    cp0 = pltpu.async_copy(h_hbm.at[hi_v], he_v, sem0)
    cp1 = pltpu.async_copy(h_hbm.at[ti_v], te_v, sem1)
    cp2 = pltpu.async_copy(rel_hbm.at[ri_v], re_v, sem2)
    cp0.wait()
    cp1.wait()
    cp2.wait()
    iota16 = lax.iota(jnp.int32, 16)
    for grp in range(2):
        sv = _z16()
        for i in range(16):
            t = grp * 16 + i
            acc = _z16()
            for hh in range(8):
                acc = acc + (he_v[t, pl.ds(hh * 16, 16)]
                             * re_v[t, pl.ds(hh * 16, 16)]
                             * te_v[t, pl.ds(hh * 16, 16)])
            stot = jnp.sum(acc)
            sv = jnp.where(iota16 == i, jnp.broadcast_to(stot, (16,)), sv)
        sc_v[pl.ds(grp * 16, 16)] = sv
    pltpu.sync_copy(sc_v, out_hbm.at[pl.ds(off, 32)])


def kernel(heads, relations, tails, edge_index, edge_type, entity_emb,
           relation_emb, comp1, bases1, root1, bias1, comp2, bases2, root2,
           bias2):
    src = edge_index[0]
    dst = edge_index[1]
    padn = EPAD - E
    src_p = jnp.concatenate([src, jnp.zeros((padn,), jnp.int32)])
    dst_p = jnp.concatenate([dst, jnp.full((padn,), N, jnp.int32)])
    et_p = jnp.concatenate([edge_type, jnp.zeros((padn,), jnp.int32)])
    comp_cat = jnp.concatenate([comp1[:, 0], comp1[:, 1],
                                comp2[:, 0], comp2[:, 1]])

    fm1, fm2 = _stage1(dst_p, et_p, comp_cat)

    src3 = src_p.reshape(16, NCH, 128)
    dst3 = dst_p.reshape(16, NCH, 128)
    imeta = jnp.stack([2 * src3, 2 * src3 + 1, dst3], axis=2)
    meta = []
    for fm_l in (fm1, fm2):
        fi = lax.bitcast_convert_type(fm_l.reshape(16, NCH, 2, 128), jnp.int32)
        m5 = jnp.concatenate([imeta, fi], axis=2)
        meta.append(m5.reshape(16, NCH // 8, 8, 5, 128))
    xpad = jnp.pad(entity_emb, ((0, NPAD - N), (0, 0)))

    h = xpad
    layer_params = (
        (0, bases1, root1, bias1.reshape(1, D), True),
        (1, bases2, root2, bias2.reshape(1, D), False),
    )
    for l, bases, root, bias2d, relu in layer_params:
        xh = h.reshape(NPAD, 2, HD).reshape(2 * NPAD, HD)
        ccat = _edge_pass(xh, meta[l])
        w0 = jnp.concatenate([bases[0][:HD, :], bases[1][:HD, :]], axis=0)
        w1 = jnp.concatenate([bases[0][HD:, :], bases[1][HD:, :]], axis=0)
        h = _dense(ccat[:NPAD], ccat[NPAD:], h, w0, w1, root, bias2d, relu)

    return _score(h, relation_emb, heads, relations, tails)
